# baseline (speedup 1.0000x reference)
import functools
import math

import jax
import jax.numpy as jnp
import numpy as np
from jax import lax
from jax.experimental import pallas as pl
from jax.experimental.pallas import tpu as pltpu
from jax.experimental.pallas import tpu_sc as plsc

N_LEVELS = 16
F_PER = 2
LOG2_T = 19
T = 1 << LOG2_T
MASK = T - 1
BASE_RES = 16
SCALE = 1.5
N_PTS = 65536
N_NEURONS = 64
PRIMES = (1, 2654435761, 805459861, 3674653429)
P1 = np.uint32(PRIMES[1]).astype(np.int32)
P2 = np.uint32(PRIMES[2]).astype(np.int32)
RES = [int(math.floor(BASE_RES * SCALE ** l)) for l in range(N_LEVELS)]

NC, NS = 2, 16
NW = NC * NS
PPW = N_PTS // NW
K = 128
NV = K // 16
NCHUNK = PPW // K
PLANE = N_LEVELS * T * F_PER


def _copy_body(tin_ref, tout_ref):
    tout_ref[...] = tin_ref[...]


def _copy_plane(flat):
    B = 1 << 16
    return pl.pallas_call(
        _copy_body,
        grid=(PLANE // B,),
        in_specs=[pl.BlockSpec((1, B), lambda i: (0, i))],
        out_specs=pl.BlockSpec((1, B), lambda i: (0, i)),
        out_shape=jax.ShapeDtypeStruct((1, PLANE), jnp.float32),
        compiler_params=pltpu.CompilerParams(
            dimension_semantics=("arbitrary",)),
    )(flat)


def _enc_body(xT, tabx, tabt, twt, tc0, tc1, out,
              xyz_v, ia0e, ia0o, ia1e, ia1o, ib0e, ib0o, ib1e, ib1o,
              ra00, ra01, ra10, ra11, rb00, rb01, rb10, rb11,
              enc_v, twt_v, tc0_v, tc1_v, sem0, sem1):
    wid = lax.axis_index("s") * NC + lax.axis_index("c")
    tx = tabx.at[0]
    tt = tabt.at[0]
    pltpu.sync_copy(twt, twt_v)
    pltpu.sync_copy(tc0, tc0_v)
    pltpu.sync_copy(tc1, tc1_v)
    ias = ((ia0e, ia0o), (ia1e, ia1o))
    ibs = ((ib0e, ib0o), (ib1e, ib1o))
    ras = ((ra00, ra01), (ra10, ra11))
    rbs = ((rb00, rb01), (rb10, rb11))
    sems = (sem0, sem1)

    def pass1(lvl):
        p = lvl & 1
        (iae, iao), (ibe, ibo) = ias[p], ibs[p]
        res = float(RES[lvl])
        c0v = tc0_v[lvl]
        c1v = tc1_v[lvl]
        lvlT = lvl * T

        def body(j, _):
            s = j * 16
            px = xyz_v[0, pl.ds(s, 16)] * res
            py = xyz_v[1, pl.ds(s, 16)] * res
            pz = xyz_v[2, pl.ds(s, 16)] * res
            ix = px.astype(jnp.int32)
            iy = py.astype(jnp.int32)
            iz = pz.astype(jnp.int32)
            txs = (ix, ix + 1)
            ty0 = iy * P1
            tys = (ty0, ty0 + P1)
            tz0 = iz * P2
            tzs = (tz0, tz0 + P2)
            for c in range(8):
                h = txs[c & 1] ^ tys[(c >> 1) & 1] ^ tzs[(c >> 2) & 1]
                da = ((h & MASK) + lvlT) * 2
                iae[pl.ds(c * K + s, 16)] = da
                iao[pl.ds(c * K + s, 16)] = da + 1
                db0 = (((h ^ c0v) & MASK) + lvlT) * 2
                ibe[pl.ds(c * K + s, 16)] = db0
                ibo[pl.ds(c * K + s, 16)] = db0 + 1
                db1 = (((h ^ c1v) & MASK) + lvlT) * 2
                ibe[pl.ds((c + 8) * K + s, 16)] = db1
                ibo[pl.ds((c + 8) * K + s, 16)] = db1 + 1
            return 0

        lax.fori_loop(0, NV, body, 0)

    def pass2(lvl):
        p = lvl & 1
        ra, rb = ras[p], rbs[p]
        res = float(RES[lvl])
        wt1 = twt_v[lvl]
        wt0 = 1.0 - wt1
        r0 = lvl * 2

        def body(j, _):
            s = j * 16
            px = xyz_v[0, pl.ds(s, 16)] * res
            py = xyz_v[1, pl.ds(s, 16)] * res
            pz = xyz_v[2, pl.ds(s, 16)] * res
            ix = px.astype(jnp.int32)
            iy = py.astype(jnp.int32)
            iz = pz.astype(jnp.int32)
            fx = px - ix.astype(jnp.float32)
            fy = py - iy.astype(jnp.float32)
            fz = pz - iz.astype(jnp.float32)
            gx = 1.0 - fx
            gy = 1.0 - fy
            gz = 1.0 - fz
            accx0 = jnp.zeros((16,), jnp.float32)
            accx1 = jnp.zeros((16,), jnp.float32)
            acct0 = jnp.zeros((16,), jnp.float32)
            acct1 = jnp.zeros((16,), jnp.float32)
            for c in range(8):
                wsp = (fx if c & 1 else gx) * (fy if c & 2 else gy) \
                    * (fz if c & 4 else gz)
                f0 = ra[0][pl.ds(c * K + s, 16)]
                f1 = ra[1][pl.ds(c * K + s, 16)]
                accx0 = accx0 + wsp * f0
                accx1 = accx1 + wsp * f1
                w0 = wsp * wt0
                w1 = wsp * wt1
                g00 = rb[0][pl.ds(c * K + s, 16)]
                g01 = rb[1][pl.ds(c * K + s, 16)]
                g10 = rb[0][pl.ds((c + 8) * K + s, 16)]
                g11 = rb[1][pl.ds((c + 8) * K + s, 16)]
                acct0 = acct0 + w0 * g00 + w1 * g10
                acct1 = acct1 + w0 * g01 + w1 * g11
            enc_v[r0, pl.ds(s, 16)] = accx0
            enc_v[r0 + 1, pl.ds(s, 16)] = accx1
            enc_v[32 + r0, pl.ds(s, 16)] = acct0
            enc_v[33 + r0, pl.ds(s, 16)] = acct1
            return 0

        lax.fori_loop(0, NV, body, 0)

    def chunk_body(ci, _):
        base = wid * PPW + ci * K
        pltpu.sync_copy(xT.at[:, pl.ds(base, K)], xyz_v)
        copies = {}
        for lvl in range(N_LEVELS):
            p = lvl & 1
            pass1(lvl)
            ca0 = pltpu.async_copy(tx.at[ias[p][0]], ras[p][0], sems[p])
            ca1 = pltpu.async_copy(tx.at[ias[p][1]], ras[p][1], sems[p])
            cb0 = pltpu.async_copy(tt.at[ibs[p][0]], rbs[p][0], sems[p])
            cb1 = pltpu.async_copy(tt.at[ibs[p][1]], rbs[p][1], sems[p])
            copies[lvl] = (ca0, ca1, cb0, cb1)
            if lvl > 0:
                for cp in copies.pop(lvl - 1):
                    cp.wait()
                pass2(lvl - 1)
        for cp in copies.pop(N_LEVELS - 1):
            cp.wait()
        pass2(N_LEVELS - 1)
        pltpu.sync_copy(enc_v, out.at[:, pl.ds(base, K)])
        return 0

    lax.fori_loop(0, NCHUNK, chunk_body, 0)


@functools.partial(jax.jit, static_argnames=())
def _encode(xT, tabx, tabt, twt, tc0, tc1):
    mesh = plsc.VectorSubcoreMesh(core_axis_name="c", subcore_axis_name="s")
    f = pl.kernel(
        _enc_body,
        out_type=jax.ShapeDtypeStruct((64, N_PTS), jnp.float32),
        mesh=mesh,
        scratch_types=[
            pltpu.VMEM((3, K), jnp.float32),
            pltpu.VMEM((8 * K,), jnp.int32),
            pltpu.VMEM((8 * K,), jnp.int32),
            pltpu.VMEM((8 * K,), jnp.int32),
            pltpu.VMEM((8 * K,), jnp.int32),
            pltpu.VMEM((16 * K,), jnp.int32),
            pltpu.VMEM((16 * K,), jnp.int32),
            pltpu.VMEM((16 * K,), jnp.int32),
            pltpu.VMEM((16 * K,), jnp.int32),
            pltpu.VMEM((8 * K,), jnp.float32),
            pltpu.VMEM((8 * K,), jnp.float32),
            pltpu.VMEM((8 * K,), jnp.float32),
            pltpu.VMEM((8 * K,), jnp.float32),
            pltpu.VMEM((16 * K,), jnp.float32),
            pltpu.VMEM((16 * K,), jnp.float32),
            pltpu.VMEM((16 * K,), jnp.float32),
            pltpu.VMEM((16 * K,), jnp.float32),
            pltpu.VMEM((64, K), jnp.float32),
            pltpu.VMEM((N_LEVELS, 16), jnp.float32),
            pltpu.VMEM((N_LEVELS, 16), jnp.int32),
            pltpu.VMEM((N_LEVELS, 16), jnp.int32),
            pltpu.SemaphoreType.DMA,
            pltpu.SemaphoreType.DMA,
        ],
    )
    return f(xT, tabx, tabt, twt, tc0, tc1)


def _mlp_body(enc_ref, w1x, w2x, w3x, w1t, w2t, w3t, out_ref):
    e = enc_ref[...]
    ex = e[:32, :]
    et = e[32:, :]
    dn = (((0,), (0,)), ((), ()))
    hp = dict(preferred_element_type=jnp.float32,
              precision=lax.Precision.HIGHEST)
    h = jnp.maximum(lax.dot_general(w1x[...], ex, dn, **hp), 0.0)
    h = jnp.maximum(lax.dot_general(w2x[...], h, dn, **hp), 0.0)
    ox = lax.dot_general(w3x[...], h, dn, **hp)
    h = jnp.maximum(lax.dot_general(w1t[...], et, dn, **hp), 0.0)
    h = jnp.maximum(lax.dot_general(w2t[...], h, dn, **hp), 0.0)
    ot = lax.dot_general(w3t[...], h, dn, **hp)
    out_ref[...] = ox + ot


def _mlp(enc, W1x, W2x, W3x, W1t, W2t, W3t):
    C = 2048
    grid = (N_PTS // C,)
    return pl.pallas_call(
        _mlp_body,
        grid=grid,
        in_specs=[
            pl.BlockSpec((64, C), lambda i: (0, i)),
            pl.BlockSpec((32, 64), lambda i: (0, 0)),
            pl.BlockSpec((64, 64), lambda i: (0, 0)),
            pl.BlockSpec((64, 1), lambda i: (0, 0)),
            pl.BlockSpec((32, 64), lambda i: (0, 0)),
            pl.BlockSpec((64, 64), lambda i: (0, 0)),
            pl.BlockSpec((64, 1), lambda i: (0, 0)),
        ],
        out_specs=pl.BlockSpec((1, C), lambda i: (0, i)),
        out_shape=jax.ShapeDtypeStruct((1, N_PTS), jnp.float32),
    )(enc, W1x, W2x, W3x, W1t, W2t, W3t)


def kernel(x, t, table_x, W1x, W2x, W3x, table_t, W1t, W2t, W3t):
    xT = x.T
    tabx = _copy_plane(table_x.reshape(1, PLANE))
    tabt = _copy_plane(table_t.reshape(1, PLANE))
    res_arr = jnp.asarray([float(r) for r in RES], jnp.float32)
    post = jnp.asarray(t, jnp.float32) * res_arr
    t0 = jnp.floor(post)
    wt = post - t0
    t0u = t0.astype(jnp.uint32)
    c0 = lax.bitcast_convert_type(t0u * jnp.uint32(PRIMES[3]), jnp.int32)
    c1 = lax.bitcast_convert_type((t0u + 1) * jnp.uint32(PRIMES[3]), jnp.int32)
    twt = jnp.broadcast_to(wt[:, None], (N_LEVELS, 16))
    tc0 = jnp.broadcast_to(c0[:, None], (N_LEVELS, 16))
    tc1 = jnp.broadcast_to(c1[:, None], (N_LEVELS, 16))
    enc = _encode(xT, tabx, tabt, twt, tc0, tc1)
    out = _mlp(enc, W1x, W2x, W3x, W1t, W2t, W3t)
    return out.reshape(N_PTS, 1)


# speedup vs baseline: 1.7274x; 1.7274x over previous
import functools
import math

import jax
import jax.numpy as jnp
import numpy as np
from jax import lax
from jax.experimental import pallas as pl
from jax.experimental.pallas import tpu as pltpu
from jax.experimental.pallas import tpu_sc as plsc

N_LEVELS = 16
F_PER = 2
LOG2_T = 19
T = 1 << LOG2_T
MASK = T - 1
BASE_RES = 16
SCALE = 1.5
N_PTS = 65536
N_NEURONS = 64
PRIMES = (1, 2654435761, 805459861, 3674653429)
P1 = np.uint32(PRIMES[1]).astype(np.int32)
P2 = np.uint32(PRIMES[2]).astype(np.int32)
RES = [int(math.floor(BASE_RES * SCALE ** l)) for l in range(N_LEVELS)]

NC, NS = 2, 16
NW = NC * NS
PPW = N_PTS // NW
K = 256
NV = K // 16
NCHUNK = PPW // K


def _fmt_body(tin_ref, t0_ref, t1_ref):
    v = tin_ref[0]
    sel = jnp.eye(2, dtype=jnp.float32)
    dn = (((1,), (1,)), ((), ()))
    planes = lax.dot_general(sel, v, dn, preferred_element_type=jnp.float32,
                             precision=lax.Precision.HIGHEST)
    t0_ref[...] = planes[0:1, :]
    t1_ref[...] = planes[1:2, :]


def _format_table(tab):
    B = 16384
    NB = T // B
    return pl.pallas_call(
        _fmt_body,
        grid=(N_LEVELS, NB),
        in_specs=[pl.BlockSpec((1, B, F_PER), lambda l, j: (l, j, 0))],
        out_specs=[
            pl.BlockSpec((1, B), lambda l, j: (0, l * NB + j)),
            pl.BlockSpec((1, B), lambda l, j: (0, l * NB + j)),
        ],
        out_shape=[
            jax.ShapeDtypeStruct((1, N_LEVELS * T), jnp.float32),
            jax.ShapeDtypeStruct((1, N_LEVELS * T), jnp.float32),
        ],
        compiler_params=pltpu.CompilerParams(
            dimension_semantics=("parallel", "arbitrary")),
    )(tab)


def _enc_body(xT, tabx0, tabx1, tabt0, tabt1, twt, tc0, tc1, out,
              xyz_v, ia0, ia1, ib0, ib1,
              ra00, ra01, ra10, ra11, rb00, rb01, rb10, rb11,
              enc_v, twt_v, tc0_v, tc1_v, sem0, sem1):
    wid = lax.axis_index("s") * NC + lax.axis_index("c")
    tx0 = tabx0.at[0]
    tx1 = tabx1.at[0]
    tt0 = tabt0.at[0]
    tt1 = tabt1.at[0]
    pltpu.sync_copy(twt, twt_v)
    pltpu.sync_copy(tc0, tc0_v)
    pltpu.sync_copy(tc1, tc1_v)
    ias = (ia0, ia1)
    ibs = (ib0, ib1)
    ras = ((ra00, ra01), (ra10, ra11))
    rbs = ((rb00, rb01), (rb10, rb11))
    sems = (sem0, sem1)

    def pass1(lvl):
        p = lvl & 1
        ia, ib = ias[p], ibs[p]
        res = float(RES[lvl])
        c0v = tc0_v[lvl]
        c1v = tc1_v[lvl]
        lvlT = lvl * T

        def body(j, _):
            s = j * 16
            px = xyz_v[0, pl.ds(s, 16)] * res
            py = xyz_v[1, pl.ds(s, 16)] * res
            pz = xyz_v[2, pl.ds(s, 16)] * res
            ix = px.astype(jnp.int32)
            iy = py.astype(jnp.int32)
            iz = pz.astype(jnp.int32)
            txs = (ix, ix + 1)
            ty0 = iy * P1
            tys = (ty0, ty0 + P1)
            tz0 = iz * P2
            tzs = (tz0, tz0 + P2)
            for c in range(8):
                h = txs[c & 1] ^ tys[(c >> 1) & 1] ^ tzs[(c >> 2) & 1]
                ia[pl.ds(c * K + s, 16)] = (h & MASK) + lvlT
                ib[pl.ds(c * K + s, 16)] = ((h ^ c0v) & MASK) + lvlT
                ib[pl.ds((c + 8) * K + s, 16)] = ((h ^ c1v) & MASK) + lvlT
            return 0

        lax.fori_loop(0, NV, body, 0)

    def pass2(lvl):
        p = lvl & 1
        ra, rb = ras[p], rbs[p]
        res = float(RES[lvl])
        wt1 = twt_v[lvl]
        wt0 = 1.0 - wt1
        r0 = lvl * 2

        def body(j, _):
            s = j * 16
            px = xyz_v[0, pl.ds(s, 16)] * res
            py = xyz_v[1, pl.ds(s, 16)] * res
            pz = xyz_v[2, pl.ds(s, 16)] * res
            ix = px.astype(jnp.int32)
            iy = py.astype(jnp.int32)
            iz = pz.astype(jnp.int32)
            fx = px - ix.astype(jnp.float32)
            fy = py - iy.astype(jnp.float32)
            fz = pz - iz.astype(jnp.float32)
            gx = 1.0 - fx
            gy = 1.0 - fy
            gz = 1.0 - fz
            accx0 = jnp.zeros((16,), jnp.float32)
            accx1 = jnp.zeros((16,), jnp.float32)
            acct0 = jnp.zeros((16,), jnp.float32)
            acct1 = jnp.zeros((16,), jnp.float32)
            for c in range(8):
                wsp = (fx if c & 1 else gx) * (fy if c & 2 else gy) \
                    * (fz if c & 4 else gz)
                f0 = ra[0][pl.ds(c * K + s, 16)]
                f1 = ra[1][pl.ds(c * K + s, 16)]
                accx0 = accx0 + wsp * f0
                accx1 = accx1 + wsp * f1
                w0 = wsp * wt0
                w1 = wsp * wt1
                g00 = rb[0][pl.ds(c * K + s, 16)]
                g01 = rb[1][pl.ds(c * K + s, 16)]
                g10 = rb[0][pl.ds((c + 8) * K + s, 16)]
                g11 = rb[1][pl.ds((c + 8) * K + s, 16)]
                acct0 = acct0 + w0 * g00 + w1 * g10
                acct1 = acct1 + w0 * g01 + w1 * g11
            enc_v[r0, pl.ds(s, 16)] = accx0
            enc_v[r0 + 1, pl.ds(s, 16)] = accx1
            enc_v[32 + r0, pl.ds(s, 16)] = acct0
            enc_v[33 + r0, pl.ds(s, 16)] = acct1
            return 0

        lax.fori_loop(0, NV, body, 0)

    def chunk_body(ci, _):
        base = wid * PPW + ci * K
        pltpu.sync_copy(xT.at[:, pl.ds(base, K)], xyz_v)
        copies = {}
        for lvl in range(N_LEVELS):
            p = lvl & 1
            pass1(lvl)
            ca0 = pltpu.async_copy(tx0.at[ias[p]], ras[p][0], sems[p])
            ca1 = pltpu.async_copy(tx1.at[ias[p]], ras[p][1], sems[p])
            cb0 = pltpu.async_copy(tt0.at[ibs[p]], rbs[p][0], sems[p])
            cb1 = pltpu.async_copy(tt1.at[ibs[p]], rbs[p][1], sems[p])
            copies[lvl] = (ca0, ca1, cb0, cb1)
            if lvl > 0:
                for cp in copies.pop(lvl - 1):
                    cp.wait()
                pass2(lvl - 1)
        for cp in copies.pop(N_LEVELS - 1):
            cp.wait()
        pass2(N_LEVELS - 1)
        pltpu.sync_copy(enc_v, out.at[:, pl.ds(base, K)])
        return 0

    lax.fori_loop(0, NCHUNK, chunk_body, 0)


@functools.partial(jax.jit, static_argnames=())
def _encode(xT, tabx0, tabx1, tabt0, tabt1, twt, tc0, tc1):
    mesh = plsc.VectorSubcoreMesh(core_axis_name="c", subcore_axis_name="s")
    f = pl.kernel(
        _enc_body,
        out_type=jax.ShapeDtypeStruct((64, N_PTS), jnp.float32),
        mesh=mesh,
        scratch_types=[
            pltpu.VMEM((3, K), jnp.float32),
            pltpu.VMEM((8 * K,), jnp.int32),
            pltpu.VMEM((8 * K,), jnp.int32),
            pltpu.VMEM((16 * K,), jnp.int32),
            pltpu.VMEM((16 * K,), jnp.int32),
            pltpu.VMEM((8 * K,), jnp.float32),
            pltpu.VMEM((8 * K,), jnp.float32),
            pltpu.VMEM((8 * K,), jnp.float32),
            pltpu.VMEM((8 * K,), jnp.float32),
            pltpu.VMEM((16 * K,), jnp.float32),
            pltpu.VMEM((16 * K,), jnp.float32),
            pltpu.VMEM((16 * K,), jnp.float32),
            pltpu.VMEM((16 * K,), jnp.float32),
            pltpu.VMEM((64, K), jnp.float32),
            pltpu.VMEM((N_LEVELS, 16), jnp.float32),
            pltpu.VMEM((N_LEVELS, 16), jnp.int32),
            pltpu.VMEM((N_LEVELS, 16), jnp.int32),
            pltpu.SemaphoreType.DMA,
            pltpu.SemaphoreType.DMA,
        ],
    )
    return f(xT, tabx0, tabx1, tabt0, tabt1, twt, tc0, tc1)


def _mlp_body(enc_ref, w1x, w2x, w3x, w1t, w2t, w3t, out_ref):
    e = enc_ref[...]
    ex = e[:32, :]
    et = e[32:, :]
    dn = (((0,), (0,)), ((), ()))
    hp = dict(preferred_element_type=jnp.float32,
              precision=lax.Precision.HIGHEST)
    h = jnp.maximum(lax.dot_general(w1x[...], ex, dn, **hp), 0.0)
    h = jnp.maximum(lax.dot_general(w2x[...], h, dn, **hp), 0.0)
    ox = lax.dot_general(w3x[...], h, dn, **hp)
    h = jnp.maximum(lax.dot_general(w1t[...], et, dn, **hp), 0.0)
    h = jnp.maximum(lax.dot_general(w2t[...], h, dn, **hp), 0.0)
    ot = lax.dot_general(w3t[...], h, dn, **hp)
    out_ref[...] = ox + ot


def _mlp(enc, W1x, W2x, W3x, W1t, W2t, W3t):
    C = 2048
    grid = (N_PTS // C,)
    return pl.pallas_call(
        _mlp_body,
        grid=grid,
        in_specs=[
            pl.BlockSpec((64, C), lambda i: (0, i)),
            pl.BlockSpec((32, 64), lambda i: (0, 0)),
            pl.BlockSpec((64, 64), lambda i: (0, 0)),
            pl.BlockSpec((64, 1), lambda i: (0, 0)),
            pl.BlockSpec((32, 64), lambda i: (0, 0)),
            pl.BlockSpec((64, 64), lambda i: (0, 0)),
            pl.BlockSpec((64, 1), lambda i: (0, 0)),
        ],
        out_specs=pl.BlockSpec((1, C), lambda i: (0, i)),
        out_shape=jax.ShapeDtypeStruct((1, N_PTS), jnp.float32),
    )(enc, W1x, W2x, W3x, W1t, W2t, W3t)


def kernel(x, t, table_x, W1x, W2x, W3x, table_t, W1t, W2t, W3t):
    xT = x.T
    tabx0, tabx1 = _format_table(table_x)
    tabt0, tabt1 = _format_table(table_t)
    res_arr = jnp.asarray([float(r) for r in RES], jnp.float32)
    post = jnp.asarray(t, jnp.float32) * res_arr
    t0 = jnp.floor(post)
    wt = post - t0
    t0u = t0.astype(jnp.uint32)
    c0 = lax.bitcast_convert_type(t0u * jnp.uint32(PRIMES[3]), jnp.int32)
    c1 = lax.bitcast_convert_type((t0u + 1) * jnp.uint32(PRIMES[3]), jnp.int32)
    twt = jnp.broadcast_to(wt[:, None], (N_LEVELS, 16))
    tc0 = jnp.broadcast_to(c0[:, None], (N_LEVELS, 16))
    tc1 = jnp.broadcast_to(c1[:, None], (N_LEVELS, 16))
    enc = _encode(xT, tabx0, tabx1, tabt0, tabt1, twt, tc0, tc1)
    out = _mlp(enc, W1x, W2x, W3x, W1t, W2t, W3t)
    return out.reshape(N_PTS, 1)
